# baseline (speedup 1.0000x reference)
import jax
import jax.numpy as jnp
from jax.experimental import pallas as pl
from jax.experimental.pallas import tpu as pltpu


def _fused_kernel(x_ref, w_ref, adj_ref, adj2_ref, out_ref, s_ref):
    @pl.when(pl.program_id(0) == 0)
    def _():
        s_ref[...] = jnp.dot(
            x_ref[...].astype(jnp.bfloat16),
            w_ref[...].astype(jnp.bfloat16),
            preferred_element_type=jnp.float32,
        ).astype(jnp.bfloat16)

    half = adj_ref.shape[0]
    acc0 = jnp.dot(
        adj_ref[...].astype(jnp.bfloat16),
        s_ref[...],
        preferred_element_type=jnp.float32,
    )
    out_ref[:half, :] = jnp.maximum(acc0, 0.0)
    acc1 = jnp.dot(
        adj2_ref[...].astype(jnp.bfloat16),
        s_ref[...],
        preferred_element_type=jnp.float32,
    )
    out_ref[half:, :] = jnp.maximum(acc1, 0.0)


def kernel(input, adj, W):
    n, d_in = input.shape
    d_out = W.shape[1]

    bm = 400
    out = pl.pallas_call(
        _fused_kernel,
        grid=(n // bm,),
        in_specs=[
            pl.BlockSpec((n, d_in), lambda i: (0, 0)),
            pl.BlockSpec((d_in, d_out), lambda i: (0, 0)),
            pl.BlockSpec((bm // 2, n), lambda i: (2 * i, 0)),
            pl.BlockSpec((bm // 2, n), lambda i: (2 * i + 1, 0)),
        ],
        out_specs=pl.BlockSpec((bm, d_out), lambda i: (i, 0)),
        out_shape=jax.ShapeDtypeStruct((n, d_out), jnp.float32),
        scratch_shapes=[
            pltpu.VMEM((n, d_out), jnp.bfloat16),
        ],
    )(input, W, adj, adj)
    return out
